# baseline (device time: 27913 ns/iter reference)
import jax
import jax.numpy as jnp
from jax import lax
from jax.experimental import pallas as pl
from jax.experimental.pallas import tpu as pltpu

N_DEV = 4
B, Sq, D = 2, 256, 768
HQ, DH = 8, 64
DL = HQ * DH
CH = (B * Sq) // N_DEV
SCALE = 0.125
BF = jnp.bfloat16


def kernel(x, Wq, Wo, Wk, Wv):
    def body(x_ref, wq_ref, wo_ref, wk_ref, wv_ref, out_ref,
             x_v, wq_v, wo_v, wk_v, wv_v, outv_ref,
             q_ref, k_ref, v_ref, attnc_ref, pb_ref, stage_ref, own_ref,
             recv_ref, ag_ref,
             in_sems, out_sems, sc_send_sems, sc_recv_sems, ag_send_sems,
             ag_recv_sems):
        my = lax.axis_index("i")
        others = [(my + 1) % N_DEV, (my + 2) % N_DEV, (my + 3) % N_DEV]

        cp_x = pltpu.make_async_copy(x_ref, x_v, in_sems.at[0])
        cp_x.start()
        cp_wq = pltpu.make_async_copy(wq_ref, wq_v, in_sems.at[1])
        cp_wq.start()

        barrier_sem = pltpu.get_barrier_semaphore()
        for nbr in others:
            pl.semaphore_signal(
                barrier_sem, inc=1,
                device_id=(nbr,), device_id_type=pl.DeviceIdType.MESH,
            )

        cp_x.wait()
        cp_wk = pltpu.make_async_copy(wk_ref, wk_v, in_sems.at[2])
        cp_wk.start()
        cp_wv = pltpu.make_async_copy(wv_ref, wv_v, in_sems.at[3])
        cp_wv.start()
        cp_wo = pltpu.make_async_copy(wo_ref, wo_v, in_sems.at[4])
        cp_wo.start()

        xb = x_v[:].reshape(B * Sq, D).astype(BF)
        cp_wq.wait()
        q_ref[:] = (jnp.dot(xb, wq_v[:].astype(BF),
                            preferred_element_type=jnp.float32)
                    * SCALE).astype(BF)
        cp_wk.wait()
        k_ref[:] = jnp.dot(xb, wk_v[:].astype(BF),
                           preferred_element_type=jnp.float32).astype(BF)
        cp_wv.wait()
        v_ref[:] = jnp.dot(xb, wv_v[:].astype(BF),
                           preferred_element_type=jnp.float32).astype(BF)
        cp_wo.wait()
        wo_b = wo_v[:].astype(BF)

        rows = lax.broadcasted_iota(jnp.int32, (HQ * Sq, HQ), 0)
        cols = lax.broadcasted_iota(jnp.int32, (HQ * Sq, HQ), 1)
        blockones = jnp.where(rows // Sq == cols, 1.0, 0.0).astype(BF)

        pl.semaphore_wait(barrier_sem, N_DEV - 1)

        sends = []
        for kk in range(N_DEV):
            c = (my + 1 + kk) % N_DEV
            qc = q_ref[pl.ds(c * CH, CH), :]
            b0 = (c // 2) * Sq
            kb = k_ref[pl.ds(b0, Sq), :]
            vb = v_ref[pl.ds(b0, Sq), :]
            for h in range(HQ):
                s = lax.dot_general(
                    qc[:, h * DH:(h + 1) * DH], kb[:, h * DH:(h + 1) * DH],
                    (((1,), (1,)), ((), ())),
                    preferred_element_type=jnp.float32,
                )
                pb_ref[:, h * Sq:(h + 1) * Sq] = jnp.exp(s).astype(BF)
            pb = pb_ref[:]
            linv = 1.0 / jnp.dot(pb, blockones,
                                 preferred_element_type=jnp.float32)
            for h in range(HQ):
                c0 = h * DH
                o = jnp.dot(pb[:, h * Sq:(h + 1) * Sq], vb[:, c0:c0 + DH],
                            preferred_element_type=jnp.float32)
                attnc_ref[:, c0:c0 + DH] = (o * linv[:, h:h + 1]).astype(BF)
            partial = jnp.dot(attnc_ref[:], wo_b,
                              preferred_element_type=jnp.float32)
            if kk < N_DEV - 1:
                stage_ref[kk] = partial.astype(BF)
                r = (N_DEV - 1) - kk
                rdma = pltpu.make_async_remote_copy(
                    src_ref=stage_ref.at[kk],
                    dst_ref=recv_ref.at[r],
                    send_sem=sc_send_sems.at[kk],
                    recv_sem=sc_recv_sems.at[r],
                    device_id=(c,),
                    device_id_type=pl.DeviceIdType.MESH,
                )
                rdma.start()
                sends.append(rdma)
            else:
                own_ref[:] = partial

        acc = own_ref[:]
        for r in (3, 2, 1):
            rd = pltpu.make_async_remote_copy(
                src_ref=stage_ref.at[0],
                dst_ref=recv_ref.at[r],
                send_sem=sc_send_sems.at[0],
                recv_sem=sc_recv_sems.at[r],
                device_id=(my,),
                device_id_type=pl.DeviceIdType.MESH,
            )
            rd.wait_recv()
            acc = acc + recv_ref[r].astype(jnp.float32)
        ag_ref[my] = acc.astype(BF)

        ag_sends = []
        for t in range(1, N_DEV):
            peer = (my + t) % N_DEV
            rdma = pltpu.make_async_remote_copy(
                src_ref=ag_ref.at[my],
                dst_ref=ag_ref.at[my],
                send_sem=ag_send_sems.at[t - 1],
                recv_sem=ag_recv_sems.at[N_DEV - t],
                device_id=(peer,),
                device_id_type=pl.DeviceIdType.MESH,
            )
            rdma.start()
            ag_sends.append(rdma)

        out_cps = []
        mb, ms = my // 2, (my % 2) * CH
        outv_ref[mb, pl.ds(ms, CH), :] = acc
        cp = pltpu.make_async_copy(
            outv_ref.at[mb, pl.ds(ms, CH), :],
            out_ref.at[mb, pl.ds(ms, CH), :],
            out_sems.at[0],
        )
        cp.start()
        out_cps.append(cp)

        for s_ in sends:
            s_.wait_send()

        for s in range(1, N_DEV):
            owner = (my + s) % N_DEV
            rd = pltpu.make_async_remote_copy(
                src_ref=ag_ref.at[0],
                dst_ref=ag_ref.at[owner],
                send_sem=ag_send_sems.at[0],
                recv_sem=ag_recv_sems.at[s],
                device_id=(my,),
                device_id_type=pl.DeviceIdType.MESH,
            )
            rd.wait_recv()
            ob, os_ = owner // 2, (owner % 2) * CH
            outv_ref[ob, pl.ds(os_, CH), :] = ag_ref[owner].astype(
                jnp.float32)
            cp = pltpu.make_async_copy(
                outv_ref.at[ob, pl.ds(os_, CH), :],
                out_ref.at[ob, pl.ds(os_, CH), :],
                out_sems.at[s],
            )
            cp.start()
            out_cps.append(cp)

        for cp in out_cps:
            cp.wait()
        for s_ in ag_sends:
            s_.wait_send()

    return pl.pallas_call(
        body,
        out_shape=jax.ShapeDtypeStruct((B, Sq, D), jnp.float32),
        in_specs=[
            pl.BlockSpec(memory_space=pltpu.MemorySpace.HBM),
            pl.BlockSpec(memory_space=pltpu.MemorySpace.HBM),
            pl.BlockSpec(memory_space=pltpu.MemorySpace.HBM),
            pl.BlockSpec(memory_space=pltpu.MemorySpace.HBM),
            pl.BlockSpec(memory_space=pltpu.MemorySpace.HBM),
        ],
        out_specs=pl.BlockSpec(memory_space=pltpu.MemorySpace.HBM),
        scratch_shapes=[
            pltpu.VMEM((B, Sq, D), jnp.float32),
            pltpu.VMEM((D, DL), jnp.float32),
            pltpu.VMEM((DL, D), jnp.float32),
            pltpu.VMEM((D, DL), jnp.float32),
            pltpu.VMEM((D, DL), jnp.float32),
            pltpu.VMEM((B, Sq, D), jnp.float32),
            pltpu.VMEM((B * Sq, DL), BF),
            pltpu.VMEM((B * Sq, DL), BF),
            pltpu.VMEM((B * Sq, DL), BF),
            pltpu.VMEM((CH, DL), BF),
            pltpu.VMEM((CH, HQ * Sq), BF),
            pltpu.VMEM((N_DEV - 1, CH, D), BF),
            pltpu.VMEM((CH, D), jnp.float32),
            pltpu.VMEM((N_DEV, CH, D), BF),
            pltpu.VMEM((N_DEV, CH, D), BF),
            pltpu.SemaphoreType.DMA((5,)),
            pltpu.SemaphoreType.DMA((N_DEV,)),
            pltpu.SemaphoreType.DMA((N_DEV - 1,)),
            pltpu.SemaphoreType.DMA((N_DEV,)),
            pltpu.SemaphoreType.DMA((N_DEV - 1,)),
            pltpu.SemaphoreType.DMA((N_DEV,)),
        ],
        compiler_params=pltpu.CompilerParams(collective_id=0),
    )(x, Wq, Wo, Wk, Wv)


# device time: 26938 ns/iter; 1.0362x vs baseline; 1.0362x over previous
import jax
import jax.numpy as jnp
from jax import lax
from jax.experimental import pallas as pl
from jax.experimental.pallas import tpu as pltpu

N_DEV = 4
B, Sq, D = 2, 256, 768
HQ, DH = 8, 64
DL = HQ * DH
CH = (B * Sq) // N_DEV
SCALE = 0.125
BF = jnp.bfloat16


def kernel(x, Wq, Wo, Wk, Wv):
    x, Wq, Wo, Wk, Wv = (a.astype(BF) for a in (x, Wq, Wo, Wk, Wv))

    def body(x_ref, wq_ref, wo_ref, wk_ref, wv_ref, out_ref,
             x_v, wq_v, wo_v, wk_v, wv_v, outv_ref,
             q_ref, k_ref, v_ref, attnc_ref, pb_ref, stage_ref, own_ref,
             recv_ref, ag_ref,
             in_sems, out_sems, sc_send_sems, sc_recv_sems, ag_send_sems,
             ag_recv_sems):
        my = lax.axis_index("i")
        others = [(my + 1) % N_DEV, (my + 2) % N_DEV, (my + 3) % N_DEV]

        cp_x = pltpu.make_async_copy(x_ref, x_v, in_sems.at[0])
        cp_x.start()
        cp_wq = pltpu.make_async_copy(wq_ref, wq_v, in_sems.at[1])
        cp_wq.start()

        barrier_sem = pltpu.get_barrier_semaphore()
        for nbr in others:
            pl.semaphore_signal(
                barrier_sem, inc=1,
                device_id=(nbr,), device_id_type=pl.DeviceIdType.MESH,
            )

        cp_x.wait()
        cp_wk = pltpu.make_async_copy(wk_ref, wk_v, in_sems.at[2])
        cp_wk.start()
        cp_wv = pltpu.make_async_copy(wv_ref, wv_v, in_sems.at[3])
        cp_wv.start()
        cp_wo = pltpu.make_async_copy(wo_ref, wo_v, in_sems.at[4])
        cp_wo.start()

        xb = x_v[:].reshape(B * Sq, D)
        cp_wq.wait()
        q_ref[:] = (jnp.dot(xb, wq_v[:],
                            preferred_element_type=jnp.float32)
                    * SCALE).astype(BF)
        cp_wk.wait()
        k_ref[:] = jnp.dot(xb, wk_v[:],
                           preferred_element_type=jnp.float32).astype(BF)
        cp_wv.wait()
        v_ref[:] = jnp.dot(xb, wv_v[:],
                           preferred_element_type=jnp.float32).astype(BF)
        cp_wo.wait()
        wo_b = wo_v[:]

        rows = lax.broadcasted_iota(jnp.int32, (HQ * Sq, HQ), 0)
        cols = lax.broadcasted_iota(jnp.int32, (HQ * Sq, HQ), 1)
        blockones = jnp.where(rows // Sq == cols, 1.0, 0.0).astype(BF)

        pl.semaphore_wait(barrier_sem, N_DEV - 1)

        sends = []
        for kk in range(N_DEV):
            c = (my + 1 + kk) % N_DEV
            qc = q_ref[pl.ds(c * CH, CH), :]
            b0 = (c // 2) * Sq
            kb = k_ref[pl.ds(b0, Sq), :]
            vb = v_ref[pl.ds(b0, Sq), :]
            for h in range(HQ):
                s = lax.dot_general(
                    qc[:, h * DH:(h + 1) * DH], kb[:, h * DH:(h + 1) * DH],
                    (((1,), (1,)), ((), ())),
                    preferred_element_type=jnp.float32,
                )
                pb_ref[:, h * Sq:(h + 1) * Sq] = jnp.exp(s).astype(BF)
            pb = pb_ref[:]
            linv = 1.0 / jnp.dot(pb, blockones,
                                 preferred_element_type=jnp.float32)
            for h in range(HQ):
                c0 = h * DH
                o = jnp.dot(pb[:, h * Sq:(h + 1) * Sq], vb[:, c0:c0 + DH],
                            preferred_element_type=jnp.float32)
                attnc_ref[:, c0:c0 + DH] = (o * linv[:, h:h + 1]).astype(BF)
            partial = jnp.dot(attnc_ref[:], wo_b,
                              preferred_element_type=jnp.float32)
            if kk < N_DEV - 1:
                stage_ref[kk] = partial.astype(BF)
                r = (N_DEV - 1) - kk
                rdma = pltpu.make_async_remote_copy(
                    src_ref=stage_ref.at[kk],
                    dst_ref=recv_ref.at[r],
                    send_sem=sc_send_sems.at[kk],
                    recv_sem=sc_recv_sems.at[r],
                    device_id=(c,),
                    device_id_type=pl.DeviceIdType.MESH,
                )
                rdma.start()
                sends.append(rdma)
            else:
                own_ref[:] = partial

        acc = own_ref[:]
        for r in (3, 2, 1):
            rd = pltpu.make_async_remote_copy(
                src_ref=stage_ref.at[0],
                dst_ref=recv_ref.at[r],
                send_sem=sc_send_sems.at[0],
                recv_sem=sc_recv_sems.at[r],
                device_id=(my,),
                device_id_type=pl.DeviceIdType.MESH,
            )
            rd.wait_recv()
            acc = acc + recv_ref[r].astype(jnp.float32)
        ag_ref[my] = acc.astype(BF)

        ag_sends = []
        for t in range(1, N_DEV):
            peer = (my + t) % N_DEV
            rdma = pltpu.make_async_remote_copy(
                src_ref=ag_ref.at[my],
                dst_ref=ag_ref.at[my],
                send_sem=ag_send_sems.at[t - 1],
                recv_sem=ag_recv_sems.at[N_DEV - t],
                device_id=(peer,),
                device_id_type=pl.DeviceIdType.MESH,
            )
            rdma.start()
            ag_sends.append(rdma)

        out_cps = []
        mb, ms = my // 2, (my % 2) * CH
        outv_ref[mb, pl.ds(ms, CH), :] = acc
        cp = pltpu.make_async_copy(
            outv_ref.at[mb, pl.ds(ms, CH), :],
            out_ref.at[mb, pl.ds(ms, CH), :],
            out_sems.at[0],
        )
        cp.start()
        out_cps.append(cp)

        for s_ in sends:
            s_.wait_send()

        for s in range(1, N_DEV):
            owner = (my + s) % N_DEV
            rd = pltpu.make_async_remote_copy(
                src_ref=ag_ref.at[0],
                dst_ref=ag_ref.at[owner],
                send_sem=ag_send_sems.at[0],
                recv_sem=ag_recv_sems.at[s],
                device_id=(my,),
                device_id_type=pl.DeviceIdType.MESH,
            )
            rd.wait_recv()
            ob, os_ = owner // 2, (owner % 2) * CH
            outv_ref[ob, pl.ds(os_, CH), :] = ag_ref[owner].astype(
                jnp.float32)
            cp = pltpu.make_async_copy(
                outv_ref.at[ob, pl.ds(os_, CH), :],
                out_ref.at[ob, pl.ds(os_, CH), :],
                out_sems.at[s],
            )
            cp.start()
            out_cps.append(cp)

        for cp in out_cps:
            cp.wait()
        for s_ in ag_sends:
            s_.wait_send()

    return pl.pallas_call(
        body,
        out_shape=jax.ShapeDtypeStruct((B, Sq, D), jnp.float32),
        in_specs=[
            pl.BlockSpec(memory_space=pltpu.MemorySpace.HBM),
            pl.BlockSpec(memory_space=pltpu.MemorySpace.HBM),
            pl.BlockSpec(memory_space=pltpu.MemorySpace.HBM),
            pl.BlockSpec(memory_space=pltpu.MemorySpace.HBM),
            pl.BlockSpec(memory_space=pltpu.MemorySpace.HBM),
        ],
        out_specs=pl.BlockSpec(memory_space=pltpu.MemorySpace.HBM),
        scratch_shapes=[
            pltpu.VMEM((B, Sq, D), BF),
            pltpu.VMEM((D, DL), BF),
            pltpu.VMEM((DL, D), BF),
            pltpu.VMEM((D, DL), BF),
            pltpu.VMEM((D, DL), BF),
            pltpu.VMEM((B, Sq, D), jnp.float32),
            pltpu.VMEM((B * Sq, DL), BF),
            pltpu.VMEM((B * Sq, DL), BF),
            pltpu.VMEM((B * Sq, DL), BF),
            pltpu.VMEM((CH, DL), BF),
            pltpu.VMEM((CH, HQ * Sq), BF),
            pltpu.VMEM((N_DEV - 1, CH, D), BF),
            pltpu.VMEM((CH, D), jnp.float32),
            pltpu.VMEM((N_DEV, CH, D), BF),
            pltpu.VMEM((N_DEV, CH, D), BF),
            pltpu.SemaphoreType.DMA((5,)),
            pltpu.SemaphoreType.DMA((N_DEV,)),
            pltpu.SemaphoreType.DMA((N_DEV - 1,)),
            pltpu.SemaphoreType.DMA((N_DEV,)),
            pltpu.SemaphoreType.DMA((N_DEV - 1,)),
            pltpu.SemaphoreType.DMA((N_DEV,)),
        ],
        compiler_params=pltpu.CompilerParams(collective_id=0),
    )(x, Wq, Wo, Wk, Wv)


# device time: 25115 ns/iter; 1.1114x vs baseline; 1.0726x over previous
import jax
import jax.numpy as jnp
from jax import lax
from jax.experimental import pallas as pl
from jax.experimental.pallas import tpu as pltpu

N_DEV = 4
B, Sq, D = 2, 256, 768
HQ, DH = 8, 64
DL = HQ * DH
CH = (B * Sq) // N_DEV
SCALE = 0.125
BF = jnp.bfloat16


def kernel(x, Wq, Wo, Wk, Wv):
    xb16 = x.astype(BF)
    W = jnp.concatenate(
        [Wq.astype(BF), Wk.astype(BF), Wv.astype(BF), Wo.T.astype(BF)],
        axis=1)

    def body(x_ref, w_ref, out_ref,
             x_v, wq_v, wo_v, wk_v, wv_v, outv_ref,
             q_ref, k_ref, v_ref, attnc_ref, pb_ref, stage_ref, own_ref,
             recv_ref, ag_ref,
             in_sems, out_sems, sc_send_sems, sc_recv_sems, ag_send_sems,
             ag_recv_sems):
        my = lax.axis_index("i")
        others = [(my + 1) % N_DEV, (my + 2) % N_DEV, (my + 3) % N_DEV]

        cp_x = pltpu.make_async_copy(x_ref, x_v, in_sems.at[0])
        cp_x.start()
        cp_wq = pltpu.make_async_copy(
            w_ref.at[:, 0:DL], wq_v, in_sems.at[1])
        cp_wq.start()

        barrier_sem = pltpu.get_barrier_semaphore()
        for nbr in others:
            pl.semaphore_signal(
                barrier_sem, inc=1,
                device_id=(nbr,), device_id_type=pl.DeviceIdType.MESH,
            )

        cp_x.wait()
        cp_wk = pltpu.make_async_copy(
            w_ref.at[:, DL:2 * DL], wk_v, in_sems.at[2])
        cp_wk.start()
        cp_wv = pltpu.make_async_copy(
            w_ref.at[:, 2 * DL:3 * DL], wv_v, in_sems.at[3])
        cp_wv.start()
        cp_wo = pltpu.make_async_copy(
            w_ref.at[:, 3 * DL:4 * DL], wo_v, in_sems.at[4])
        cp_wo.start()

        xb = x_v[:].reshape(B * Sq, D)
        cp_wq.wait()
        q_ref[:] = (jnp.dot(xb, wq_v[:],
                            preferred_element_type=jnp.float32)
                    * SCALE).astype(BF)
        cp_wk.wait()
        k_ref[:] = jnp.dot(xb, wk_v[:],
                           preferred_element_type=jnp.float32).astype(BF)
        cp_wv.wait()
        v_ref[:] = jnp.dot(xb, wv_v[:],
                           preferred_element_type=jnp.float32).astype(BF)
        cp_wo.wait()
        wo_t = wo_v[:]

        rows = lax.broadcasted_iota(jnp.int32, (HQ * Sq, HQ), 0)
        cols = lax.broadcasted_iota(jnp.int32, (HQ * Sq, HQ), 1)
        blockones = jnp.where(rows // Sq == cols, 1.0, 0.0).astype(BF)

        pl.semaphore_wait(barrier_sem, N_DEV - 1)

        sends = []
        for kk in range(N_DEV):
            c = (my + 1 + kk) % N_DEV
            qc = q_ref[pl.ds(c * CH, CH), :]
            b0 = (c // 2) * Sq
            kb = k_ref[pl.ds(b0, Sq), :]
            vb = v_ref[pl.ds(b0, Sq), :]
            for h in range(HQ):
                s = lax.dot_general(
                    qc[:, h * DH:(h + 1) * DH], kb[:, h * DH:(h + 1) * DH],
                    (((1,), (1,)), ((), ())),
                    preferred_element_type=jnp.float32,
                )
                pb_ref[:, h * Sq:(h + 1) * Sq] = jnp.exp(s).astype(BF)
            pb = pb_ref[:]
            linv = 1.0 / jnp.dot(pb, blockones,
                                 preferred_element_type=jnp.float32)
            for h in range(HQ):
                c0 = h * DH
                o = jnp.dot(pb[:, h * Sq:(h + 1) * Sq], vb[:, c0:c0 + DH],
                            preferred_element_type=jnp.float32)
                attnc_ref[:, c0:c0 + DH] = (o * linv[:, h:h + 1]).astype(BF)
            partial = lax.dot_general(
                attnc_ref[:], wo_t, (((1,), (1,)), ((), ())),
                preferred_element_type=jnp.float32)
            if kk < N_DEV - 1:
                stage_ref[kk] = partial.astype(BF)
                r = (N_DEV - 1) - kk
                rdma = pltpu.make_async_remote_copy(
                    src_ref=stage_ref.at[kk],
                    dst_ref=recv_ref.at[r],
                    send_sem=sc_send_sems.at[kk],
                    recv_sem=sc_recv_sems.at[r],
                    device_id=(c,),
                    device_id_type=pl.DeviceIdType.MESH,
                )
                rdma.start()
                sends.append(rdma)
            else:
                own_ref[:] = partial

        acc = own_ref[:]
        for r in (3, 2, 1):
            rd = pltpu.make_async_remote_copy(
                src_ref=stage_ref.at[0],
                dst_ref=recv_ref.at[r],
                send_sem=sc_send_sems.at[0],
                recv_sem=sc_recv_sems.at[r],
                device_id=(my,),
                device_id_type=pl.DeviceIdType.MESH,
            )
            rd.wait_recv()
            acc = acc + recv_ref[r].astype(jnp.float32)
        ag_ref[my] = acc.astype(BF)

        ag_sends = []
        for t in range(1, N_DEV):
            peer = (my + t) % N_DEV
            rdma = pltpu.make_async_remote_copy(
                src_ref=ag_ref.at[my],
                dst_ref=ag_ref.at[my],
                send_sem=ag_send_sems.at[t - 1],
                recv_sem=ag_recv_sems.at[N_DEV - t],
                device_id=(peer,),
                device_id_type=pl.DeviceIdType.MESH,
            )
            rdma.start()
            ag_sends.append(rdma)

        out_cps = []
        mb, ms = my // 2, (my % 2) * CH
        outv_ref[mb, pl.ds(ms, CH), :] = acc
        cp = pltpu.make_async_copy(
            outv_ref.at[mb, pl.ds(ms, CH), :],
            out_ref.at[mb, pl.ds(ms, CH), :],
            out_sems.at[0],
        )
        cp.start()
        out_cps.append(cp)

        for s_ in sends:
            s_.wait_send()

        for s in range(1, N_DEV):
            owner = (my + s) % N_DEV
            rd = pltpu.make_async_remote_copy(
                src_ref=ag_ref.at[0],
                dst_ref=ag_ref.at[owner],
                send_sem=ag_send_sems.at[0],
                recv_sem=ag_recv_sems.at[s],
                device_id=(my,),
                device_id_type=pl.DeviceIdType.MESH,
            )
            rd.wait_recv()
            ob, os_ = owner // 2, (owner % 2) * CH
            outv_ref[ob, pl.ds(os_, CH), :] = ag_ref[owner].astype(
                jnp.float32)
            cp = pltpu.make_async_copy(
                outv_ref.at[ob, pl.ds(os_, CH), :],
                out_ref.at[ob, pl.ds(os_, CH), :],
                out_sems.at[s],
            )
            cp.start()
            out_cps.append(cp)

        for cp in out_cps:
            cp.wait()
        for s_ in ag_sends:
            s_.wait_send()

    return pl.pallas_call(
        body,
        out_shape=jax.ShapeDtypeStruct((B, Sq, D), jnp.float32),
        in_specs=[
            pl.BlockSpec(memory_space=pltpu.MemorySpace.HBM),
            pl.BlockSpec(memory_space=pltpu.MemorySpace.HBM),
        ],
        out_specs=pl.BlockSpec(memory_space=pltpu.MemorySpace.HBM),
        scratch_shapes=[
            pltpu.VMEM((B, Sq, D), BF),
            pltpu.VMEM((D, DL), BF),
            pltpu.VMEM((D, DL), BF),
            pltpu.VMEM((D, DL), BF),
            pltpu.VMEM((D, DL), BF),
            pltpu.VMEM((B, Sq, D), jnp.float32),
            pltpu.VMEM((B * Sq, DL), BF),
            pltpu.VMEM((B * Sq, DL), BF),
            pltpu.VMEM((B * Sq, DL), BF),
            pltpu.VMEM((CH, DL), BF),
            pltpu.VMEM((CH, HQ * Sq), BF),
            pltpu.VMEM((N_DEV - 1, CH, D), BF),
            pltpu.VMEM((CH, D), jnp.float32),
            pltpu.VMEM((N_DEV, CH, D), BF),
            pltpu.VMEM((N_DEV, CH, D), BF),
            pltpu.SemaphoreType.DMA((5,)),
            pltpu.SemaphoreType.DMA((N_DEV,)),
            pltpu.SemaphoreType.DMA((N_DEV - 1,)),
            pltpu.SemaphoreType.DMA((N_DEV,)),
            pltpu.SemaphoreType.DMA((N_DEV - 1,)),
            pltpu.SemaphoreType.DMA((N_DEV,)),
        ],
        compiler_params=pltpu.CompilerParams(collective_id=0),
    )(xb16, W)
